# Initial kernel scaffold; baseline (speedup 1.0000x reference)
import jax
import jax.numpy as jnp
from jax.experimental import pallas as pl


def kernel(user_emb, item_emb, W1, b1, W2, b2, edge_index):
    raise NotImplementedError("write your pallas kernel here")


# speedup vs baseline: 7.7352x; 7.7352x over previous
import functools

import jax
import jax.numpy as jnp
from jax import lax
from jax.experimental import pallas as pl
from jax.experimental.pallas import tpu as pltpu
from jax.experimental.pallas import tpu_sc as plsc

N_U = 50000
N_I = 50000
N = N_U + N_I
D = 128
EH = 300000
NC = 2
NS = 16
CH = 128
CPT = 147
EP = NS * CPT * CH
TPT = CPT * CH
GARB = N_U
ACC_ROWS = 50048
STRIPE = 3128
STRIPE_LAST = 50000 - 15 * STRIPE
BN = 1000
NBLK = N // BN


def _sc_mesh():
    return plsc.VectorSubcoreMesh(
        core_axis_name="c", subcore_axis_name="s",
        num_cores=NC, num_subcores=NS)


@functools.cache
def _sc_degree_call():
    return pl.kernel(
        _sc_degree_body,
        out_type=jax.ShapeDtypeStruct((N, 16), jnp.float32),
        mesh=_sc_mesh(),
        scratch_types=[
            pltpu.VMEM_SHARED((ACC_ROWS, 16), jnp.float32),
            pltpu.VMEM((CPT, CH), jnp.int32),
            pltpu.VMEM((CH, 16), jnp.float32),
        ],
        compiler_params=pltpu.CompilerParams(use_tc_tiling_on_sc=False),
    )


def _sc_degree_body(dst4, zrows, ones, deg_out, acc, didx, onesv):
    c = lax.axis_index("c")
    s = lax.axis_index("s")
    pltpu.sync_copy(zrows, acc.at[pl.ds(s * STRIPE, STRIPE)])
    pltpu.sync_copy(ones, onesv)
    pltpu.sync_copy(dst4.at[c, s], didx)
    plsc.subcore_barrier()

    def body(j, carry):
        pltpu.sync_copy(onesv, acc.at[didx.at[j]], add=True)
        return carry

    lax.fori_loop(0, CPT, body, 0)
    plsc.subcore_barrier()
    base = (1 - c) * N_U

    @pl.when(s < NS - 1)
    def _():
        pltpu.sync_copy(acc.at[pl.ds(s * STRIPE, STRIPE)],
                        deg_out.at[pl.ds(base + s * STRIPE, STRIPE)])

    @pl.when(s == NS - 1)
    def _():
        pltpu.sync_copy(acc.at[pl.ds((NS - 1) * STRIPE, STRIPE_LAST)],
                        deg_out.at[pl.ds(base + (NS - 1) * STRIPE, STRIPE_LAST)])


@functools.cache
def _sc_aggregate_call():
    return pl.kernel(
        _sc_aggregate_body,
        out_type=[jax.ShapeDtypeStruct((N, 32), jnp.float32) for _ in range(4)],
        mesh=_sc_mesh(),
        scratch_types=[
            pltpu.VMEM_SHARED((ACC_ROWS, 32), jnp.float32),
            pltpu.VMEM((4, CH), jnp.int32),
            pltpu.VMEM((4, CH), jnp.int32),
            [pltpu.VMEM((CH, 32), jnp.float32) for _ in range(4)],
            [pltpu.SemaphoreType.DMA for _ in range(4)],
        ],
        compiler_params=pltpu.CompilerParams(use_tc_tiling_on_sc=False),
    )


def _sc_aggregate_body(src4, dst4, z0, z1, z2, z3, zrows,
                       a0, a1, a2, a3, acc, sidx, didx, bufs, sems):
    c = lax.axis_index("c")
    s = lax.axis_index("s")
    base = (1 - c) * N_U
    zs = (z0, z1, z2, z3)
    outs = (a0, a1, a2, a3)
    for cc in range(4):
        zc = zs[cc]
        pltpu.sync_copy(zrows, acc.at[pl.ds(s * STRIPE, STRIPE)])
        plsc.subcore_barrier()

        def quad(j4, carry, zc=zc):
            j = j4 * 4
            pltpu.sync_copy(src4.at[c, s, pl.ds(j, 4)], sidx)
            pltpu.sync_copy(dst4.at[c, s, pl.ds(j, 4)], didx)
            cps = [
                pltpu.async_copy(zc.at[sidx.at[k]], bufs[k], sems[k])
                for k in range(4)
            ]
            for k in range(4):
                cps[k].wait()
                pltpu.sync_copy(bufs[k], acc.at[didx.at[k]], add=True)
            return carry

        lax.fori_loop(0, CPT // 4, quad, 0)
        j = (CPT // 4) * 4
        pltpu.sync_copy(src4.at[c, s, pl.ds(j, CPT % 4)], sidx.at[pl.ds(0, CPT % 4)])
        pltpu.sync_copy(dst4.at[c, s, pl.ds(j, CPT % 4)], didx.at[pl.ds(0, CPT % 4)])
        for k in range(CPT % 4):
            pltpu.async_copy(zc.at[sidx.at[k]], bufs[k], sems[k]).wait()
            pltpu.sync_copy(bufs[k], acc.at[didx.at[k]], add=True)
        plsc.subcore_barrier()

        @pl.when(s < NS - 1)
        def _(cc=cc):
            pltpu.sync_copy(acc.at[pl.ds(s * STRIPE, STRIPE)],
                            outs[cc].at[pl.ds(base + s * STRIPE, STRIPE)])

        @pl.when(s == NS - 1)
        def _(cc=cc):
            pltpu.sync_copy(
                acc.at[pl.ds((NS - 1) * STRIPE, STRIPE_LAST)],
                outs[cc].at[pl.ds(base + (NS - 1) * STRIPE, STRIPE_LAST)])


def _dinv(deg_blk):
    return lax.rsqrt(deg_blk[:, 0:1] + 1.0)


def _tc_prep_body(x_ref, deg_ref, w_ref, z0, z1, z2, z3):
    xw = jnp.dot(x_ref[:], w_ref[:], preferred_element_type=jnp.float32)
    z = xw * _dinv(deg_ref[:])
    z0[:] = z[:, 0:32]
    z1[:] = z[:, 32:64]
    z2[:] = z[:, 64:96]
    z3[:] = z[:, 96:128]


_tc_prep = pl.pallas_call(
    _tc_prep_body,
    grid=(NBLK,),
    in_specs=[
        pl.BlockSpec((BN, D), lambda i: (i, 0)),
        pl.BlockSpec((BN, 16), lambda i: (i, 0)),
        pl.BlockSpec((D, D), lambda i: (0, 0)),
    ],
    out_specs=[pl.BlockSpec((BN, 32), lambda i: (i, 0)) for _ in range(4)],
    out_shape=[jax.ShapeDtypeStruct((N, 32), jnp.float32) for _ in range(4)],
)


def _tc_mid_body(x_ref, deg_ref, w_ref, b_ref,
                 z0, z1, z2, z3, a0, a1, a2, a3,
                 y0, y1, y2, y3, part_ref):
    dinv = _dinv(deg_ref[:])
    zagg = jnp.concatenate(
        [z0[:] + a0[:], z1[:] + a1[:], z2[:] + a2[:], z3[:] + a3[:]], axis=1)
    ego1 = dinv * zagg + b_ref[:]
    part_ref[:] = x_ref[:] + ego1
    xw = jnp.dot(ego1, w_ref[:], preferred_element_type=jnp.float32)
    z = xw * dinv
    y0[:] = z[:, 0:32]
    y1[:] = z[:, 32:64]
    y2[:] = z[:, 64:96]
    y3[:] = z[:, 96:128]


_tc_mid = pl.pallas_call(
    _tc_mid_body,
    grid=(NBLK,),
    in_specs=[
        pl.BlockSpec((BN, D), lambda i: (i, 0)),
        pl.BlockSpec((BN, 16), lambda i: (i, 0)),
        pl.BlockSpec((D, D), lambda i: (0, 0)),
        pl.BlockSpec((1, D), lambda i: (0, 0)),
    ] + [pl.BlockSpec((BN, 32), lambda i: (i, 0)) for _ in range(8)],
    out_specs=[pl.BlockSpec((BN, 32), lambda i: (i, 0)) for _ in range(4)]
    + [pl.BlockSpec((BN, D), lambda i: (i, 0))],
    out_shape=[jax.ShapeDtypeStruct((N, 32), jnp.float32) for _ in range(4)]
    + [jax.ShapeDtypeStruct((N, D), jnp.float32)],
)


def _final_ego2(deg_ref, b_ref, chunks):
    dinv = _dinv(deg_ref[:])
    z0, z1, z2, z3, a0, a1, a2, a3 = chunks
    zagg = jnp.concatenate(
        [z0[:] + a0[:], z1[:] + a1[:], z2[:] + a2[:], z3[:] + a3[:]], axis=1)
    return dinv * zagg + b_ref[:]


def _tc_final_u_body(part_ref, deg_ref, b_ref, *rest):
    chunks, out_ref = rest[:8], rest[8]
    ego2 = _final_ego2(deg_ref, b_ref, chunks)
    out_ref[:] = (part_ref[:] + ego2) * (1.0 / 3.0)


def _tc_final_i_body(part_ref, deg_ref, b_ref, *rest):
    chunks, extra_ref, out_ref = rest[:8], rest[8], rest[9]
    ego2 = _final_ego2(deg_ref, b_ref, chunks)
    out_ref[:] = (part_ref[:] + ego2) * (1.0 / 3.0) + extra_ref[:]


def _make_tc_final(body, off_blk, nrows, with_extra):
    nb = nrows // BN
    specs = [
        pl.BlockSpec((BN, D), lambda i: (i + off_blk, 0)),
        pl.BlockSpec((BN, 16), lambda i: (i + off_blk, 0)),
        pl.BlockSpec((1, D), lambda i: (0, 0)),
    ] + [pl.BlockSpec((BN, 32), lambda i: (i + off_blk, 0)) for _ in range(8)]
    if with_extra:
        specs.append(pl.BlockSpec((BN, D), lambda i: (i, 0)))
    return pl.pallas_call(
        body,
        grid=(nb,),
        in_specs=specs,
        out_specs=pl.BlockSpec((BN, D), lambda i: (i, 0)),
        out_shape=jax.ShapeDtypeStruct((nrows, D), jnp.float32),
    )


_tc_final_u = _make_tc_final(_tc_final_u_body, 0, N_U, False)
_tc_final_i = _make_tc_final(_tc_final_i_body, N_U // BN, N_I, True)


def kernel(user_emb, item_emb, W1, b1, W2, b2, edge_index):
    src = edge_index[0]
    dst = edge_index[1]
    padn = EP - EH
    pad_s = jnp.zeros((padn,), jnp.int32)
    pad_d = jnp.full((padn,), GARB, jnp.int32)
    src4 = jnp.stack([
        jnp.concatenate([src[:EH], pad_s]),
        jnp.concatenate([src[EH:], pad_s]),
    ]).reshape(NC, NS, CPT, CH)
    dst4 = jnp.stack([
        jnp.concatenate([dst[:EH] - N_U, pad_d]),
        jnp.concatenate([dst[EH:], pad_d]),
    ]).reshape(NC, NS, CPT, CH)

    ego0 = jnp.concatenate([user_emb, item_emb], axis=0)
    b1r = b1.reshape(1, D)
    b2r = b2.reshape(1, D)
    zrows16 = jnp.zeros((STRIPE, 16), jnp.float32)
    zrows32 = jnp.zeros((STRIPE, 32), jnp.float32)
    ones16 = jnp.ones((CH, 16), jnp.float32)

    deg16 = _sc_degree_call()(dst4, zrows16, ones16)
    zc1 = _tc_prep(ego0, deg16, W1)
    ac1 = _sc_aggregate_call()(src4, dst4, *zc1, zrows32)
    *zc2, part = _tc_mid(ego0, deg16, W2, b1r, *zc1, *ac1)
    ac2 = _sc_aggregate_call()(src4, dst4, *zc2, zrows32)
    u_g = _tc_final_u(part, deg16, b2r, *zc2, *ac2)
    i_g = _tc_final_i(part, deg16, b2r, *zc2, *ac2, item_emb)
    return (u_g, i_g)


# speedup vs baseline: 8.9046x; 1.1512x over previous
import functools

import jax
import jax.numpy as jnp
from jax import lax
from jax.experimental import pallas as pl
from jax.experimental.pallas import tpu as pltpu
from jax.experimental.pallas import tpu_sc as plsc

N_U = 50000
N_I = 50000
N = N_U + N_I
D = 128
EH = 300000
NC = 2
NS = 16
CH = 128
CPT = 147
EP = NS * CPT * CH
TPT = CPT * CH
GARB = N_U
ACC_ROWS = 50048
STRIPE = 3128
STRIPE_LAST = 50000 - 15 * STRIPE
BN = 1000
NBLK = N // BN


def _sc_mesh():
    return plsc.VectorSubcoreMesh(
        core_axis_name="c", subcore_axis_name="s",
        num_cores=NC, num_subcores=NS)


@functools.cache
def _sc_degree_call():
    return pl.kernel(
        _sc_degree_body,
        out_type=jax.ShapeDtypeStruct((N, 16), jnp.float32),
        mesh=_sc_mesh(),
        scratch_types=[
            pltpu.VMEM_SHARED((ACC_ROWS, 16), jnp.float32),
            pltpu.VMEM((CPT, CH), jnp.int32),
            pltpu.VMEM((CH, 16), jnp.float32),
        ],
        compiler_params=pltpu.CompilerParams(use_tc_tiling_on_sc=False),
    )


def _sc_degree_body(dst4, zrows, ones, deg_out, acc, didx, onesv):
    c = lax.axis_index("c")
    s = lax.axis_index("s")
    pltpu.sync_copy(zrows, acc.at[pl.ds(s * STRIPE, STRIPE)])
    pltpu.sync_copy(ones, onesv)
    pltpu.sync_copy(dst4.at[c, s], didx)
    plsc.subcore_barrier()

    def body(j, carry):
        pltpu.sync_copy(onesv, acc.at[didx.at[j]], add=True)
        return carry

    lax.fori_loop(0, CPT, body, 0)
    plsc.subcore_barrier()
    base = (1 - c) * N_U

    @pl.when(s < NS - 1)
    def _():
        pltpu.sync_copy(acc.at[pl.ds(s * STRIPE, STRIPE)],
                        deg_out.at[pl.ds(base + s * STRIPE, STRIPE)])

    @pl.when(s == NS - 1)
    def _():
        pltpu.sync_copy(acc.at[pl.ds((NS - 1) * STRIPE, STRIPE_LAST)],
                        deg_out.at[pl.ds(base + (NS - 1) * STRIPE, STRIPE_LAST)])


@functools.cache
def _sc_aggregate_call():
    return pl.kernel(
        _sc_aggregate_body,
        out_type=[jax.ShapeDtypeStruct((N, 32), jnp.float32) for _ in range(4)],
        mesh=_sc_mesh(),
        scratch_types=[
            pltpu.VMEM_SHARED((ACC_ROWS, 32), jnp.float32),
            pltpu.VMEM((8, CH), jnp.int32),
            pltpu.VMEM((8, CH), jnp.int32),
            [pltpu.VMEM((CH, 32), jnp.float32) for _ in range(4)],
            [pltpu.SemaphoreType.DMA for _ in range(4)],
            [pltpu.SemaphoreType.DMA for _ in range(4)],
            [pltpu.SemaphoreType.DMA for _ in range(2)],
        ],
        compiler_params=pltpu.CompilerParams(use_tc_tiling_on_sc=False),
    )


_NQUAD = CPT // 4


def _sc_aggregate_body(src4, dst4, z0, z1, z2, z3, zrows,
                       a0, a1, a2, a3, acc, sidx, didx, bufs,
                       gsems, ssems, isems):
    c = lax.axis_index("c")
    s = lax.axis_index("s")
    base = (1 - c) * N_U
    zs = (z0, z1, z2, z3)
    outs = (a0, a1, a2, a3)
    for cc in range(4):
        zc = zs[cc]
        pltpu.sync_copy(zrows, acc.at[pl.ds(s * STRIPE, STRIPE)])
        plsc.subcore_barrier()
        pltpu.sync_copy(src4.at[c, s, pl.ds(0, 4)], sidx.at[pl.ds(0, 4)])
        pltpu.sync_copy(dst4.at[c, s, pl.ds(0, 4)], didx.at[pl.ds(0, 4)])

        def quad(j4, carry, zc=zc):
            off = (j4 % 2) * 4
            noff = 4 - off

            @pl.when(j4 > 0)
            def _():
                pltpu.make_async_copy(
                    src4.at[c, s, pl.ds(0, 4)], sidx.at[pl.ds(off, 4)],
                    isems[0]).wait()
                pltpu.make_async_copy(
                    dst4.at[c, s, pl.ds(0, 4)], didx.at[pl.ds(off, 4)],
                    isems[1]).wait()

            @pl.when(j4 > 0)
            def _():
                for k in range(4):
                    pltpu.make_async_copy(
                        bufs[k], acc.at[didx.at[off + k]], ssems[k]).wait()

            @pl.when(j4 < _NQUAD - 1)
            def _():
                pltpu.async_copy(src4.at[c, s, pl.ds((j4 + 1) * 4, 4)],
                                 sidx.at[pl.ds(noff, 4)], isems[0])
                pltpu.async_copy(dst4.at[c, s, pl.ds((j4 + 1) * 4, 4)],
                                 didx.at[pl.ds(noff, 4)], isems[1])

            cps = [
                pltpu.async_copy(zc.at[sidx.at[off + k]], bufs[k], gsems[k])
                for k in range(4)
            ]
            for k in range(4):
                cps[k].wait()
                pltpu.async_copy(bufs[k], acc.at[didx.at[off + k]],
                                 ssems[k], add=True)
            return carry

        lax.fori_loop(0, _NQUAD, quad, 0)
        for k in range(4):
            pltpu.make_async_copy(bufs[k], acc.at[didx.at[k]],
                                  ssems[k]).wait()
        j = _NQUAD * 4
        pltpu.sync_copy(src4.at[c, s, pl.ds(j, CPT % 4)], sidx.at[pl.ds(0, CPT % 4)])
        pltpu.sync_copy(dst4.at[c, s, pl.ds(j, CPT % 4)], didx.at[pl.ds(0, CPT % 4)])
        for k in range(CPT % 4):
            pltpu.async_copy(zc.at[sidx.at[k]], bufs[k], gsems[k]).wait()
            pltpu.sync_copy(bufs[k], acc.at[didx.at[k]], add=True)
        plsc.subcore_barrier()

        @pl.when(s < NS - 1)
        def _(cc=cc):
            pltpu.sync_copy(acc.at[pl.ds(s * STRIPE, STRIPE)],
                            outs[cc].at[pl.ds(base + s * STRIPE, STRIPE)])

        @pl.when(s == NS - 1)
        def _(cc=cc):
            pltpu.sync_copy(
                acc.at[pl.ds((NS - 1) * STRIPE, STRIPE_LAST)],
                outs[cc].at[pl.ds(base + (NS - 1) * STRIPE, STRIPE_LAST)])


def _dinv(deg_blk):
    return lax.rsqrt(deg_blk[:, 0:1] + 1.0)


def _tc_prep_body(x_ref, deg_ref, w_ref, z0, z1, z2, z3):
    xw = jnp.dot(x_ref[:], w_ref[:], preferred_element_type=jnp.float32)
    z = xw * _dinv(deg_ref[:])
    z0[:] = z[:, 0:32]
    z1[:] = z[:, 32:64]
    z2[:] = z[:, 64:96]
    z3[:] = z[:, 96:128]


_tc_prep = pl.pallas_call(
    _tc_prep_body,
    grid=(NBLK,),
    in_specs=[
        pl.BlockSpec((BN, D), lambda i: (i, 0)),
        pl.BlockSpec((BN, 16), lambda i: (i, 0)),
        pl.BlockSpec((D, D), lambda i: (0, 0)),
    ],
    out_specs=[pl.BlockSpec((BN, 32), lambda i: (i, 0)) for _ in range(4)],
    out_shape=[jax.ShapeDtypeStruct((N, 32), jnp.float32) for _ in range(4)],
)


def _tc_mid_body(x_ref, deg_ref, w_ref, b_ref,
                 z0, z1, z2, z3, a0, a1, a2, a3,
                 y0, y1, y2, y3, part_ref):
    dinv = _dinv(deg_ref[:])
    zagg = jnp.concatenate(
        [z0[:] + a0[:], z1[:] + a1[:], z2[:] + a2[:], z3[:] + a3[:]], axis=1)
    ego1 = dinv * zagg + b_ref[:]
    part_ref[:] = x_ref[:] + ego1
    xw = jnp.dot(ego1, w_ref[:], preferred_element_type=jnp.float32)
    z = xw * dinv
    y0[:] = z[:, 0:32]
    y1[:] = z[:, 32:64]
    y2[:] = z[:, 64:96]
    y3[:] = z[:, 96:128]


_tc_mid = pl.pallas_call(
    _tc_mid_body,
    grid=(NBLK,),
    in_specs=[
        pl.BlockSpec((BN, D), lambda i: (i, 0)),
        pl.BlockSpec((BN, 16), lambda i: (i, 0)),
        pl.BlockSpec((D, D), lambda i: (0, 0)),
        pl.BlockSpec((1, D), lambda i: (0, 0)),
    ] + [pl.BlockSpec((BN, 32), lambda i: (i, 0)) for _ in range(8)],
    out_specs=[pl.BlockSpec((BN, 32), lambda i: (i, 0)) for _ in range(4)]
    + [pl.BlockSpec((BN, D), lambda i: (i, 0))],
    out_shape=[jax.ShapeDtypeStruct((N, 32), jnp.float32) for _ in range(4)]
    + [jax.ShapeDtypeStruct((N, D), jnp.float32)],
)


def _final_ego2(deg_ref, b_ref, chunks):
    dinv = _dinv(deg_ref[:])
    z0, z1, z2, z3, a0, a1, a2, a3 = chunks
    zagg = jnp.concatenate(
        [z0[:] + a0[:], z1[:] + a1[:], z2[:] + a2[:], z3[:] + a3[:]], axis=1)
    return dinv * zagg + b_ref[:]


def _tc_final_u_body(part_ref, deg_ref, b_ref, *rest):
    chunks, out_ref = rest[:8], rest[8]
    ego2 = _final_ego2(deg_ref, b_ref, chunks)
    out_ref[:] = (part_ref[:] + ego2) * (1.0 / 3.0)


def _tc_final_i_body(part_ref, deg_ref, b_ref, *rest):
    chunks, extra_ref, out_ref = rest[:8], rest[8], rest[9]
    ego2 = _final_ego2(deg_ref, b_ref, chunks)
    out_ref[:] = (part_ref[:] + ego2) * (1.0 / 3.0) + extra_ref[:]


def _make_tc_final(body, off_blk, nrows, with_extra):
    nb = nrows // BN
    specs = [
        pl.BlockSpec((BN, D), lambda i: (i + off_blk, 0)),
        pl.BlockSpec((BN, 16), lambda i: (i + off_blk, 0)),
        pl.BlockSpec((1, D), lambda i: (0, 0)),
    ] + [pl.BlockSpec((BN, 32), lambda i: (i + off_blk, 0)) for _ in range(8)]
    if with_extra:
        specs.append(pl.BlockSpec((BN, D), lambda i: (i, 0)))
    return pl.pallas_call(
        body,
        grid=(nb,),
        in_specs=specs,
        out_specs=pl.BlockSpec((BN, D), lambda i: (i, 0)),
        out_shape=jax.ShapeDtypeStruct((nrows, D), jnp.float32),
    )


_tc_final_u = _make_tc_final(_tc_final_u_body, 0, N_U, False)
_tc_final_i = _make_tc_final(_tc_final_i_body, N_U // BN, N_I, True)


def kernel(user_emb, item_emb, W1, b1, W2, b2, edge_index):
    src = edge_index[0]
    dst = edge_index[1]
    padn = EP - EH
    pad_s = jnp.zeros((padn,), jnp.int32)
    pad_d = jnp.full((padn,), GARB, jnp.int32)
    src4 = jnp.stack([
        jnp.concatenate([src[:EH], pad_s]),
        jnp.concatenate([src[EH:], pad_s]),
    ]).reshape(NC, NS, CPT, CH)
    dst4 = jnp.stack([
        jnp.concatenate([dst[:EH] - N_U, pad_d]),
        jnp.concatenate([dst[EH:], pad_d]),
    ]).reshape(NC, NS, CPT, CH)

    ego0 = jnp.concatenate([user_emb, item_emb], axis=0)
    b1r = b1.reshape(1, D)
    b2r = b2.reshape(1, D)
    zrows16 = jnp.zeros((STRIPE, 16), jnp.float32)
    zrows32 = jnp.zeros((STRIPE, 32), jnp.float32)
    ones16 = jnp.ones((CH, 16), jnp.float32)

    deg16 = _sc_degree_call()(dst4, zrows16, ones16)
    zc1 = _tc_prep(ego0, deg16, W1)
    ac1 = _sc_aggregate_call()(src4, dst4, *zc1, zrows32)
    *zc2, part = _tc_mid(ego0, deg16, W2, b1r, *zc1, *ac1)
    ac2 = _sc_aggregate_call()(src4, dst4, *zc2, zrows32)
    u_g = _tc_final_u(part, deg16, b2r, *zc2, *ac2)
    i_g = _tc_final_i(part, deg16, b2r, *zc2, *ac2, item_emb)
    return (u_g, i_g)


# speedup vs baseline: 9.1242x; 1.0247x over previous
import functools

import jax
import jax.numpy as jnp
from jax import lax
from jax.experimental import pallas as pl
from jax.experimental.pallas import tpu as pltpu
from jax.experimental.pallas import tpu_sc as plsc

N_U = 50000
N_I = 50000
N = N_U + N_I
D = 128
EH = 300000
NC = 2
NS = 16
CH = 128
CPT = 147
EP = NS * CPT * CH
TPT = CPT * CH
GARB = N_U
ACC_ROWS = 50048
STRIPE = 3128
STRIPE_LAST = 50000 - 15 * STRIPE
BN = 2000
NBLK = N // BN


def _sc_mesh():
    return plsc.VectorSubcoreMesh(
        core_axis_name="c", subcore_axis_name="s",
        num_cores=NC, num_subcores=NS)


@functools.cache
def _sc_degree_call():
    return pl.kernel(
        _sc_degree_body,
        out_type=jax.ShapeDtypeStruct((N, 16), jnp.float32),
        mesh=_sc_mesh(),
        scratch_types=[
            pltpu.VMEM_SHARED((ACC_ROWS, 16), jnp.float32),
            pltpu.VMEM((CPT, CH), jnp.int32),
            pltpu.VMEM((CH, 16), jnp.float32),
        ],
        compiler_params=pltpu.CompilerParams(use_tc_tiling_on_sc=False),
    )


def _sc_degree_body(dst4, zrows, ones, deg_out, acc, didx, onesv):
    c = lax.axis_index("c")
    s = lax.axis_index("s")
    pltpu.sync_copy(zrows, acc.at[pl.ds(s * STRIPE, STRIPE)])
    pltpu.sync_copy(ones, onesv)
    pltpu.sync_copy(dst4.at[c, s], didx)
    plsc.subcore_barrier()

    def body(j, carry):
        pltpu.sync_copy(onesv, acc.at[didx.at[j]], add=True)
        return carry

    lax.fori_loop(0, CPT, body, 0)
    plsc.subcore_barrier()
    base = (1 - c) * N_U

    @pl.when(s < NS - 1)
    def _():
        pltpu.sync_copy(acc.at[pl.ds(s * STRIPE, STRIPE)],
                        deg_out.at[pl.ds(base + s * STRIPE, STRIPE)])

    @pl.when(s == NS - 1)
    def _():
        pltpu.sync_copy(acc.at[pl.ds((NS - 1) * STRIPE, STRIPE_LAST)],
                        deg_out.at[pl.ds(base + (NS - 1) * STRIPE, STRIPE_LAST)])


@functools.cache
def _sc_aggregate_call():
    return pl.kernel(
        _sc_aggregate_body,
        out_type=[jax.ShapeDtypeStruct((N, 32), jnp.float32) for _ in range(4)],
        mesh=_sc_mesh(),
        scratch_types=[
            pltpu.VMEM_SHARED((ACC_ROWS, 32), jnp.float32),
            pltpu.VMEM((8, CH), jnp.int32),
            pltpu.VMEM((8, CH), jnp.int32),
            [pltpu.VMEM((CH, 32), jnp.float32) for _ in range(4)],
            [pltpu.SemaphoreType.DMA for _ in range(4)],
            [pltpu.SemaphoreType.DMA for _ in range(4)],
            [pltpu.SemaphoreType.DMA for _ in range(2)],
        ],
        compiler_params=pltpu.CompilerParams(use_tc_tiling_on_sc=False),
    )


_NQUAD = CPT // 4


def _sc_aggregate_body(src4, dst4, z0, z1, z2, z3, zrows,
                       a0, a1, a2, a3, acc, sidx, didx, bufs,
                       gsems, ssems, isems):
    c = lax.axis_index("c")
    s = lax.axis_index("s")
    base = (1 - c) * N_U
    zs = (z0, z1, z2, z3)
    outs = (a0, a1, a2, a3)
    for cc in range(4):
        zc = zs[cc]
        pltpu.sync_copy(zrows, acc.at[pl.ds(s * STRIPE, STRIPE)])
        plsc.subcore_barrier()
        pltpu.sync_copy(src4.at[c, s, pl.ds(0, 4)], sidx.at[pl.ds(0, 4)])
        pltpu.sync_copy(dst4.at[c, s, pl.ds(0, 4)], didx.at[pl.ds(0, 4)])

        def quad(j4, carry, zc=zc):
            off = (j4 % 2) * 4
            noff = 4 - off

            @pl.when(j4 > 0)
            def _():
                pltpu.make_async_copy(
                    src4.at[c, s, pl.ds(0, 4)], sidx.at[pl.ds(off, 4)],
                    isems[0]).wait()
                pltpu.make_async_copy(
                    dst4.at[c, s, pl.ds(0, 4)], didx.at[pl.ds(off, 4)],
                    isems[1]).wait()

            @pl.when(j4 > 0)
            def _():
                for k in range(4):
                    pltpu.make_async_copy(
                        bufs[k], acc.at[didx.at[off + k]], ssems[k]).wait()

            @pl.when(j4 < _NQUAD - 1)
            def _():
                pltpu.async_copy(src4.at[c, s, pl.ds((j4 + 1) * 4, 4)],
                                 sidx.at[pl.ds(noff, 4)], isems[0])
                pltpu.async_copy(dst4.at[c, s, pl.ds((j4 + 1) * 4, 4)],
                                 didx.at[pl.ds(noff, 4)], isems[1])

            cps = [
                pltpu.async_copy(zc.at[sidx.at[off + k]], bufs[k], gsems[k])
                for k in range(4)
            ]
            for k in range(4):
                cps[k].wait()
                pltpu.async_copy(bufs[k], acc.at[didx.at[off + k]],
                                 ssems[k], add=True)
            return carry

        lax.fori_loop(0, _NQUAD, quad, 0)
        for k in range(4):
            pltpu.make_async_copy(bufs[k], acc.at[didx.at[k]],
                                  ssems[k]).wait()
        j = _NQUAD * 4
        pltpu.sync_copy(src4.at[c, s, pl.ds(j, CPT % 4)], sidx.at[pl.ds(0, CPT % 4)])
        pltpu.sync_copy(dst4.at[c, s, pl.ds(j, CPT % 4)], didx.at[pl.ds(0, CPT % 4)])
        for k in range(CPT % 4):
            pltpu.async_copy(zc.at[sidx.at[k]], bufs[k], gsems[k]).wait()
            pltpu.sync_copy(bufs[k], acc.at[didx.at[k]], add=True)
        plsc.subcore_barrier()

        @pl.when(s < NS - 1)
        def _(cc=cc):
            pltpu.sync_copy(acc.at[pl.ds(s * STRIPE, STRIPE)],
                            outs[cc].at[pl.ds(base + s * STRIPE, STRIPE)])

        @pl.when(s == NS - 1)
        def _(cc=cc):
            pltpu.sync_copy(
                acc.at[pl.ds((NS - 1) * STRIPE, STRIPE_LAST)],
                outs[cc].at[pl.ds(base + (NS - 1) * STRIPE, STRIPE_LAST)])


def _dinv(deg_blk):
    return lax.rsqrt(deg_blk[:, 0:1] + 1.0)


def _tc_prep_body(x_ref, deg_ref, w_ref, z0, z1, z2, z3):
    xw = jnp.dot(x_ref[:], w_ref[:], preferred_element_type=jnp.float32)
    z = xw * _dinv(deg_ref[:])
    z0[:] = z[:, 0:32]
    z1[:] = z[:, 32:64]
    z2[:] = z[:, 64:96]
    z3[:] = z[:, 96:128]


_tc_prep = pl.pallas_call(
    _tc_prep_body,
    grid=(NBLK,),
    in_specs=[
        pl.BlockSpec((BN, D), lambda i: (i, 0)),
        pl.BlockSpec((BN, 16), lambda i: (i, 0)),
        pl.BlockSpec((D, D), lambda i: (0, 0)),
    ],
    out_specs=[pl.BlockSpec((BN, 32), lambda i: (i, 0)) for _ in range(4)],
    out_shape=[jax.ShapeDtypeStruct((N, 32), jnp.float32) for _ in range(4)],
)


def _tc_mid_body(x_ref, deg_ref, w_ref, b_ref,
                 z0, z1, z2, z3, a0, a1, a2, a3,
                 y0, y1, y2, y3, part_ref):
    dinv = _dinv(deg_ref[:])
    zagg = jnp.concatenate(
        [z0[:] + a0[:], z1[:] + a1[:], z2[:] + a2[:], z3[:] + a3[:]], axis=1)
    ego1 = dinv * zagg + b_ref[:]
    part_ref[:] = x_ref[:] + ego1
    xw = jnp.dot(ego1, w_ref[:], preferred_element_type=jnp.float32)
    z = xw * dinv
    y0[:] = z[:, 0:32]
    y1[:] = z[:, 32:64]
    y2[:] = z[:, 64:96]
    y3[:] = z[:, 96:128]


_tc_mid = pl.pallas_call(
    _tc_mid_body,
    grid=(NBLK,),
    in_specs=[
        pl.BlockSpec((BN, D), lambda i: (i, 0)),
        pl.BlockSpec((BN, 16), lambda i: (i, 0)),
        pl.BlockSpec((D, D), lambda i: (0, 0)),
        pl.BlockSpec((1, D), lambda i: (0, 0)),
    ] + [pl.BlockSpec((BN, 32), lambda i: (i, 0)) for _ in range(8)],
    out_specs=[pl.BlockSpec((BN, 32), lambda i: (i, 0)) for _ in range(4)]
    + [pl.BlockSpec((BN, D), lambda i: (i, 0))],
    out_shape=[jax.ShapeDtypeStruct((N, 32), jnp.float32) for _ in range(4)]
    + [jax.ShapeDtypeStruct((N, D), jnp.float32)],
)


def _final_ego2(deg_ref, b_ref, chunks):
    dinv = _dinv(deg_ref[:])
    z0, z1, z2, z3, a0, a1, a2, a3 = chunks
    zagg = jnp.concatenate(
        [z0[:] + a0[:], z1[:] + a1[:], z2[:] + a2[:], z3[:] + a3[:]], axis=1)
    return dinv * zagg + b_ref[:]


def _tc_final_u_body(part_ref, deg_ref, b_ref, *rest):
    chunks, out_ref = rest[:8], rest[8]
    ego2 = _final_ego2(deg_ref, b_ref, chunks)
    out_ref[:] = (part_ref[:] + ego2) * (1.0 / 3.0)


def _tc_final_i_body(part_ref, deg_ref, b_ref, *rest):
    chunks, extra_ref, out_ref = rest[:8], rest[8], rest[9]
    ego2 = _final_ego2(deg_ref, b_ref, chunks)
    out_ref[:] = (part_ref[:] + ego2) * (1.0 / 3.0) + extra_ref[:]


def _make_tc_final(body, off_blk, nrows, with_extra):
    nb = nrows // BN
    specs = [
        pl.BlockSpec((BN, D), lambda i: (i + off_blk, 0)),
        pl.BlockSpec((BN, 16), lambda i: (i + off_blk, 0)),
        pl.BlockSpec((1, D), lambda i: (0, 0)),
    ] + [pl.BlockSpec((BN, 32), lambda i: (i + off_blk, 0)) for _ in range(8)]
    if with_extra:
        specs.append(pl.BlockSpec((BN, D), lambda i: (i, 0)))
    return pl.pallas_call(
        body,
        grid=(nb,),
        in_specs=specs,
        out_specs=pl.BlockSpec((BN, D), lambda i: (i, 0)),
        out_shape=jax.ShapeDtypeStruct((nrows, D), jnp.float32),
    )


_tc_final_u = _make_tc_final(_tc_final_u_body, 0, N_U, False)
_tc_final_i = _make_tc_final(_tc_final_i_body, N_U // BN, N_I, True)


def kernel(user_emb, item_emb, W1, b1, W2, b2, edge_index):
    src = edge_index[0]
    dst = edge_index[1]
    padn = EP - EH
    pad_s = jnp.zeros((padn,), jnp.int32)
    pad_d = jnp.full((padn,), GARB, jnp.int32)
    src4 = jnp.stack([
        jnp.concatenate([src[:EH], pad_s]),
        jnp.concatenate([src[EH:], pad_s]),
    ]).reshape(NC, NS, CPT, CH)
    dst4 = jnp.stack([
        jnp.concatenate([dst[:EH] - N_U, pad_d]),
        jnp.concatenate([dst[EH:], pad_d]),
    ]).reshape(NC, NS, CPT, CH)

    ego0 = jnp.concatenate([user_emb, item_emb], axis=0)
    b1r = b1.reshape(1, D)
    b2r = b2.reshape(1, D)
    zrows16 = jnp.zeros((STRIPE, 16), jnp.float32)
    zrows32 = jnp.zeros((STRIPE, 32), jnp.float32)
    ones16 = jnp.ones((CH, 16), jnp.float32)

    deg16 = _sc_degree_call()(dst4, zrows16, ones16)
    zc1 = _tc_prep(ego0, deg16, W1)
    ac1 = _sc_aggregate_call()(src4, dst4, *zc1, zrows32)
    *zc2, part = _tc_mid(ego0, deg16, W2, b1r, *zc1, *ac1)
    ac2 = _sc_aggregate_call()(src4, dst4, *zc2, zrows32)
    u_g = _tc_final_u(part, deg16, b2r, *zc2, *ac2)
    i_g = _tc_final_i(part, deg16, b2r, *zc2, *ac2, item_emb)
    return (u_g, i_g)


# speedup vs baseline: 9.3639x; 1.0263x over previous
import functools

import jax
import jax.numpy as jnp
from jax import lax
from jax.experimental import pallas as pl
from jax.experimental.pallas import tpu as pltpu
from jax.experimental.pallas import tpu_sc as plsc

N_U = 50000
N_I = 50000
N = N_U + N_I
D = 128
EH = 300000
NC = 2
NS = 16
CH = 128
CPT = 147
EP = NS * CPT * CH
TPT = CPT * CH
GARB = N_U
ACC_ROWS = 50048
STRIPE = 3128
STRIPE_LAST = 50000 - 15 * STRIPE
BN = 2000
NBLK = N // BN


def _sc_mesh():
    return plsc.VectorSubcoreMesh(
        core_axis_name="c", subcore_axis_name="s",
        num_cores=NC, num_subcores=NS)


@functools.cache
def _sc_degree_call():
    return pl.kernel(
        _sc_degree_body,
        out_type=jax.ShapeDtypeStruct((N, 16), jnp.float32),
        mesh=_sc_mesh(),
        scratch_types=[
            pltpu.VMEM_SHARED((ACC_ROWS, 16), jnp.float32),
            pltpu.VMEM((CPT, CH), jnp.int32),
            pltpu.VMEM((CH, 16), jnp.float32),
        ],
        compiler_params=pltpu.CompilerParams(use_tc_tiling_on_sc=False),
    )


def _sc_degree_body(dst4, zrows, ones, deg_out, acc, didx, onesv):
    c = lax.axis_index("c")
    s = lax.axis_index("s")
    pltpu.sync_copy(zrows, acc.at[pl.ds(s * STRIPE, STRIPE)])
    pltpu.sync_copy(ones, onesv)
    pltpu.sync_copy(dst4.at[c, s], didx)
    plsc.subcore_barrier()

    def body(j, carry):
        pltpu.sync_copy(onesv, acc.at[didx.at[j]], add=True)
        return carry

    lax.fori_loop(0, CPT, body, 0)
    plsc.subcore_barrier()
    base = (1 - c) * N_U

    @pl.when(s < NS - 1)
    def _():
        pltpu.sync_copy(acc.at[pl.ds(s * STRIPE, STRIPE)],
                        deg_out.at[pl.ds(base + s * STRIPE, STRIPE)])

    @pl.when(s == NS - 1)
    def _():
        pltpu.sync_copy(acc.at[pl.ds((NS - 1) * STRIPE, STRIPE_LAST)],
                        deg_out.at[pl.ds(base + (NS - 1) * STRIPE, STRIPE_LAST)])


@functools.cache
def _sc_aggregate_call():
    return pl.kernel(
        _sc_aggregate_body,
        out_type=[jax.ShapeDtypeStruct((N, 32), jnp.float32) for _ in range(4)],
        mesh=_sc_mesh(),
        scratch_types=[
            pltpu.VMEM_SHARED((ACC_ROWS, 32), jnp.float32),
            pltpu.VMEM((12, CH), jnp.int32),
            pltpu.VMEM((12, CH), jnp.int32),
            [pltpu.VMEM((CH, 32), jnp.float32) for _ in range(6)],
            [pltpu.SemaphoreType.DMA for _ in range(6)],
            [pltpu.SemaphoreType.DMA for _ in range(6)],
            [pltpu.SemaphoreType.DMA for _ in range(2)],
        ],
        compiler_params=pltpu.CompilerParams(use_tc_tiling_on_sc=False),
    )


_GRP = 6
_NQUAD = CPT // _GRP


def _sc_aggregate_body(src4, dst4, z0, z1, z2, z3, zrows,
                       a0, a1, a2, a3, acc, sidx, didx, bufs,
                       gsems, ssems, isems):
    c = lax.axis_index("c")
    s = lax.axis_index("s")
    base = (1 - c) * N_U
    zs = (z0, z1, z2, z3)
    outs = (a0, a1, a2, a3)
    for cc in range(4):
        zc = zs[cc]
        pltpu.sync_copy(zrows, acc.at[pl.ds(s * STRIPE, STRIPE)])
        plsc.subcore_barrier()
        pltpu.sync_copy(src4.at[c, s, pl.ds(0, _GRP)], sidx.at[pl.ds(0, _GRP)])
        pltpu.sync_copy(dst4.at[c, s, pl.ds(0, _GRP)], didx.at[pl.ds(0, _GRP)])

        def quad(j4, carry, zc=zc):
            off = (j4 % 2) * _GRP
            noff = _GRP - off

            @pl.when(j4 > 0)
            def _():
                pltpu.make_async_copy(
                    src4.at[c, s, pl.ds(0, _GRP)], sidx.at[pl.ds(off, _GRP)],
                    isems[0]).wait()
                pltpu.make_async_copy(
                    dst4.at[c, s, pl.ds(0, _GRP)], didx.at[pl.ds(off, _GRP)],
                    isems[1]).wait()

            @pl.when(j4 > 0)
            def _():
                for k in range(_GRP):
                    pltpu.make_async_copy(
                        bufs[k], acc.at[didx.at[off + k]], ssems[k]).wait()

            @pl.when(j4 < _NQUAD - 1)
            def _():
                pltpu.async_copy(src4.at[c, s, pl.ds((j4 + 1) * _GRP, _GRP)],
                                 sidx.at[pl.ds(noff, _GRP)], isems[0])
                pltpu.async_copy(dst4.at[c, s, pl.ds((j4 + 1) * _GRP, _GRP)],
                                 didx.at[pl.ds(noff, _GRP)], isems[1])

            cps = [
                pltpu.async_copy(zc.at[sidx.at[off + k]], bufs[k], gsems[k])
                for k in range(_GRP)
            ]
            for k in range(_GRP):
                cps[k].wait()
                pltpu.async_copy(bufs[k], acc.at[didx.at[off + k]],
                                 ssems[k], add=True)
            return carry

        lax.fori_loop(0, _NQUAD, quad, 0)
        for k in range(_GRP):
            pltpu.make_async_copy(bufs[k], acc.at[didx.at[k]],
                                  ssems[k]).wait()
        j = _NQUAD * _GRP
        pltpu.sync_copy(src4.at[c, s, pl.ds(j, CPT % 4)], sidx.at[pl.ds(0, CPT % 4)])
        pltpu.sync_copy(dst4.at[c, s, pl.ds(j, CPT % 4)], didx.at[pl.ds(0, CPT % 4)])
        for k in range(CPT % 4):
            pltpu.async_copy(zc.at[sidx.at[k]], bufs[k], gsems[k]).wait()
            pltpu.sync_copy(bufs[k], acc.at[didx.at[k]], add=True)
        plsc.subcore_barrier()

        @pl.when(s < NS - 1)
        def _(cc=cc):
            pltpu.sync_copy(acc.at[pl.ds(s * STRIPE, STRIPE)],
                            outs[cc].at[pl.ds(base + s * STRIPE, STRIPE)])

        @pl.when(s == NS - 1)
        def _(cc=cc):
            pltpu.sync_copy(
                acc.at[pl.ds((NS - 1) * STRIPE, STRIPE_LAST)],
                outs[cc].at[pl.ds(base + (NS - 1) * STRIPE, STRIPE_LAST)])


def _dinv(deg_blk):
    return lax.rsqrt(deg_blk[:, 0:1] + 1.0)


def _tc_prep_body(x_ref, deg_ref, w_ref, z0, z1, z2, z3):
    xw = jnp.dot(x_ref[:], w_ref[:], preferred_element_type=jnp.float32)
    z = xw * _dinv(deg_ref[:])
    z0[:] = z[:, 0:32]
    z1[:] = z[:, 32:64]
    z2[:] = z[:, 64:96]
    z3[:] = z[:, 96:128]


_tc_prep = pl.pallas_call(
    _tc_prep_body,
    grid=(NBLK,),
    in_specs=[
        pl.BlockSpec((BN, D), lambda i: (i, 0)),
        pl.BlockSpec((BN, 16), lambda i: (i, 0)),
        pl.BlockSpec((D, D), lambda i: (0, 0)),
    ],
    out_specs=[pl.BlockSpec((BN, 32), lambda i: (i, 0)) for _ in range(4)],
    out_shape=[jax.ShapeDtypeStruct((N, 32), jnp.float32) for _ in range(4)],
)


def _tc_mid_body(x_ref, deg_ref, w_ref, b_ref,
                 z0, z1, z2, z3, a0, a1, a2, a3,
                 y0, y1, y2, y3, part_ref):
    dinv = _dinv(deg_ref[:])
    zagg = jnp.concatenate(
        [z0[:] + a0[:], z1[:] + a1[:], z2[:] + a2[:], z3[:] + a3[:]], axis=1)
    ego1 = dinv * zagg + b_ref[:]
    part_ref[:] = x_ref[:] + ego1
    xw = jnp.dot(ego1, w_ref[:], preferred_element_type=jnp.float32)
    z = xw * dinv
    y0[:] = z[:, 0:32]
    y1[:] = z[:, 32:64]
    y2[:] = z[:, 64:96]
    y3[:] = z[:, 96:128]


_tc_mid = pl.pallas_call(
    _tc_mid_body,
    grid=(NBLK,),
    in_specs=[
        pl.BlockSpec((BN, D), lambda i: (i, 0)),
        pl.BlockSpec((BN, 16), lambda i: (i, 0)),
        pl.BlockSpec((D, D), lambda i: (0, 0)),
        pl.BlockSpec((1, D), lambda i: (0, 0)),
    ] + [pl.BlockSpec((BN, 32), lambda i: (i, 0)) for _ in range(8)],
    out_specs=[pl.BlockSpec((BN, 32), lambda i: (i, 0)) for _ in range(4)]
    + [pl.BlockSpec((BN, D), lambda i: (i, 0))],
    out_shape=[jax.ShapeDtypeStruct((N, 32), jnp.float32) for _ in range(4)]
    + [jax.ShapeDtypeStruct((N, D), jnp.float32)],
)


def _final_ego2(deg_ref, b_ref, chunks):
    dinv = _dinv(deg_ref[:])
    z0, z1, z2, z3, a0, a1, a2, a3 = chunks
    zagg = jnp.concatenate(
        [z0[:] + a0[:], z1[:] + a1[:], z2[:] + a2[:], z3[:] + a3[:]], axis=1)
    return dinv * zagg + b_ref[:]


def _tc_final_u_body(part_ref, deg_ref, b_ref, *rest):
    chunks, out_ref = rest[:8], rest[8]
    ego2 = _final_ego2(deg_ref, b_ref, chunks)
    out_ref[:] = (part_ref[:] + ego2) * (1.0 / 3.0)


def _tc_final_i_body(part_ref, deg_ref, b_ref, *rest):
    chunks, extra_ref, out_ref = rest[:8], rest[8], rest[9]
    ego2 = _final_ego2(deg_ref, b_ref, chunks)
    out_ref[:] = (part_ref[:] + ego2) * (1.0 / 3.0) + extra_ref[:]


def _make_tc_final(body, off_blk, nrows, with_extra):
    nb = nrows // BN
    specs = [
        pl.BlockSpec((BN, D), lambda i: (i + off_blk, 0)),
        pl.BlockSpec((BN, 16), lambda i: (i + off_blk, 0)),
        pl.BlockSpec((1, D), lambda i: (0, 0)),
    ] + [pl.BlockSpec((BN, 32), lambda i: (i + off_blk, 0)) for _ in range(8)]
    if with_extra:
        specs.append(pl.BlockSpec((BN, D), lambda i: (i, 0)))
    return pl.pallas_call(
        body,
        grid=(nb,),
        in_specs=specs,
        out_specs=pl.BlockSpec((BN, D), lambda i: (i, 0)),
        out_shape=jax.ShapeDtypeStruct((nrows, D), jnp.float32),
    )


_tc_final_u = _make_tc_final(_tc_final_u_body, 0, N_U, False)
_tc_final_i = _make_tc_final(_tc_final_i_body, N_U // BN, N_I, True)


def kernel(user_emb, item_emb, W1, b1, W2, b2, edge_index):
    src = edge_index[0]
    dst = edge_index[1]
    padn = EP - EH
    pad_s = jnp.zeros((padn,), jnp.int32)
    pad_d = jnp.full((padn,), GARB, jnp.int32)
    src4 = jnp.stack([
        jnp.concatenate([src[:EH], pad_s]),
        jnp.concatenate([src[EH:], pad_s]),
    ]).reshape(NC, NS, CPT, CH)
    dst4 = jnp.stack([
        jnp.concatenate([dst[:EH] - N_U, pad_d]),
        jnp.concatenate([dst[EH:], pad_d]),
    ]).reshape(NC, NS, CPT, CH)

    ego0 = jnp.concatenate([user_emb, item_emb], axis=0)
    b1r = b1.reshape(1, D)
    b2r = b2.reshape(1, D)
    zrows16 = jnp.zeros((STRIPE, 16), jnp.float32)
    zrows32 = jnp.zeros((STRIPE, 32), jnp.float32)
    ones16 = jnp.ones((CH, 16), jnp.float32)

    deg16 = _sc_degree_call()(dst4, zrows16, ones16)
    zc1 = _tc_prep(ego0, deg16, W1)
    ac1 = _sc_aggregate_call()(src4, dst4, *zc1, zrows32)
    *zc2, part = _tc_mid(ego0, deg16, W2, b1r, *zc1, *ac1)
    ac2 = _sc_aggregate_call()(src4, dst4, *zc2, zrows32)
    u_g = _tc_final_u(part, deg16, b2r, *zc2, *ac2)
    i_g = _tc_final_i(part, deg16, b2r, *zc2, *ac2, item_emb)
    return (u_g, i_g)
